# baseline (speedup 1.0000x reference)
import jax
import jax.numpy as jnp
from jax.experimental import pallas as pl
from jax.experimental.pallas import tpu as pltpu


def _body(x_ref, m_ref, o_ref, acc_s, acc_c, nlb):
    ph = pl.program_id(1)
    l = pl.program_id(2)

    @pl.when(ph == 0)
    def _accumulate():
        @pl.when(l == 0)
        def _init():
            acc_s[...] = jnp.zeros_like(acc_s)
            acc_c[...] = jnp.zeros_like(acc_c)

        m = m_ref[...].astype(jnp.float32)
        acc_s[...] += jnp.sum(x_ref[...] * m, axis=1)
        acc_c[...] += jnp.sum(m, axis=1)

    @pl.when(ph == 1)
    def _write():
        cnt = acc_c[...]
        mean = jnp.where(cnt > 0, acc_s[...] / jnp.maximum(cnt, 1.0), 0.0)
        o_ref[...] = jnp.broadcast_to(mean[:, None, :], o_ref.shape)


def kernel(x, mask):
    B, L, K, C = x.shape
    KC = K * C
    x2 = x.reshape(B, L, KC)
    m2 = mask.reshape(B, L, KC)

    LB = 1024
    nlb = L // LB

    def in_map(b, ph, l):
        return (b, l * (1 - ph) + (nlb - 1) * ph, 0)

    def out_map(b, ph, l):
        return (b, l * ph, 0)

    import functools

    out = pl.pallas_call(
        functools.partial(_body, nlb=nlb),
        out_shape=jax.ShapeDtypeStruct((B, L, KC), x.dtype),
        grid=(B, 2, nlb),
        in_specs=[
            pl.BlockSpec((1, LB, KC), in_map),
            pl.BlockSpec((1, LB, KC), in_map),
        ],
        out_specs=pl.BlockSpec((1, LB, KC), out_map),
        scratch_shapes=[
            pltpu.VMEM((1, KC), jnp.float32),
            pltpu.VMEM((1, KC), jnp.float32),
        ],
        compiler_params=pltpu.CompilerParams(
            dimension_semantics=("parallel", "arbitrary", "arbitrary"),
            vmem_limit_bytes=61 * 1024 * 1024,
        ),
        name="masked_mean_bcast",
    )(x2, m2)
    return out.reshape(B, L, K, C)
